# baseline (speedup 1.0000x reference)
import jax
import jax.numpy as jnp
from jax.experimental import pallas as pl
from jax.experimental.pallas import tpu as pltpu

B = 4096
D = 3072
H1 = 1024
ENC_OUT = 512
P1 = 256
P2 = 128
EMB = 2
EMB_PAD = 128

ROWS = 512


def _fwd_kernel(x_ref, xs_ref, w1_ref, b1_ref, w2_ref, b2_ref,
                wp1_ref, bp1_ref, wp2_ref, bp2_ref, wp3_ref, bp3_ref,
                reps_ref, emb_ref, sreps_ref, semb_ref):
    bf = jnp.bfloat16
    w1 = w1_ref[...].astype(bf)
    w2 = w2_ref[...].astype(bf)
    wp1 = wp1_ref[...].astype(bf)
    wp2 = wp2_ref[...].astype(bf)
    wp3 = wp3_ref[...].astype(bf)

    def encode(inp, reps_out, emb_out):
        h = jnp.maximum(
            jnp.dot(inp.astype(bf), w1, preferred_element_type=jnp.float32)
            + b1_ref[...], 0.0)
        reps = jnp.maximum(
            jnp.dot(h.astype(bf), w2, preferred_element_type=jnp.float32)
            + b2_ref[...], 0.0)
        reps_out[...] = reps
        e = jnp.maximum(
            jnp.dot(reps.astype(bf), wp1, preferred_element_type=jnp.float32)
            + bp1_ref[...], 0.0)
        e = jnp.maximum(
            jnp.dot(e.astype(bf), wp2, preferred_element_type=jnp.float32)
            + bp2_ref[...], 0.0)
        emb_out[...] = (jnp.dot(e.astype(bf), wp3, preferred_element_type=jnp.float32)
                        + bp3_ref[...])

    encode(x_ref[...], reps_ref, emb_ref)
    encode(xs_ref[...], sreps_ref, semb_ref)


def kernel(x, x_sim, W1, b1, W2, b2, Wp1, bp1, Wp2, bp2, Wp3, bp3):
    grid = (B // ROWS,)
    row_spec = lambda w: pl.BlockSpec((ROWS, w), lambda i: (i, 0))
    full_spec = lambda r, c: pl.BlockSpec((r, c), lambda i: (0, 0))
    bias_spec = lambda w: pl.BlockSpec((1, w), lambda i: (0, 0))

    out_shapes = (
        jax.ShapeDtypeStruct((B, ENC_OUT), jnp.float32),
        jax.ShapeDtypeStruct((B, EMB), jnp.float32),
        jax.ShapeDtypeStruct((B, ENC_OUT), jnp.float32),
        jax.ShapeDtypeStruct((B, EMB), jnp.float32),
    )

    reps, emb, sreps, semb = pl.pallas_call(
        _fwd_kernel,
        grid=grid,
        in_specs=[
            row_spec(D), row_spec(D),
            full_spec(D, H1), bias_spec(H1),
            full_spec(H1, ENC_OUT), bias_spec(ENC_OUT),
            full_spec(ENC_OUT, P1), bias_spec(P1),
            full_spec(P1, P2), bias_spec(P2),
            full_spec(P2, EMB), bias_spec(EMB),
        ],
        out_specs=(
            row_spec(ENC_OUT), row_spec(EMB),
            row_spec(ENC_OUT), row_spec(EMB),
        ),
        out_shape=out_shapes,
        compiler_params=pltpu.CompilerParams(
            dimension_semantics=("arbitrary",),
        ),
    )(x, x_sim, W1, b1.reshape(1, H1), W2, b2.reshape(1, ENC_OUT),
      Wp1, bp1.reshape(1, P1), Wp2, bp2.reshape(1, P2),
      Wp3, bp3.reshape(1, EMB))

    return (reps, emb, sreps, semb)


# speedup vs baseline: 1.0493x; 1.0493x over previous
import jax
import jax.numpy as jnp
from jax.experimental import pallas as pl
from jax.experimental.pallas import tpu as pltpu

B = 4096
D = 3072
H1 = 1024
ENC_OUT = 512
P1 = 256
P2 = 128
EMB = 2
EMB_PAD = 128

ROWS = 512


def _fwd_kernel(x_ref, xs_ref, w1_ref, b1_ref, w2_ref, b2_ref,
                wp1_ref, bp1_ref, wp2_ref, bp2_ref, wp3_ref, bp3_ref,
                reps_ref, emb_ref, sreps_ref, semb_ref):
    bf = jnp.bfloat16
    w1 = w1_ref[...].astype(bf)
    w2 = w2_ref[...].astype(bf)
    wp1 = wp1_ref[...].astype(bf)
    wp2 = wp2_ref[...].astype(bf)
    wp3 = wp3_ref[...].astype(bf)

    def layer(inp, w, b_ref):
        return jnp.maximum(
            jnp.dot(inp.astype(bf), w, preferred_element_type=jnp.float32)
            + b_ref[...], 0.0)

    hx = layer(x_ref[...], w1, b1_ref)
    hs = layer(xs_ref[...], w1, b1_ref)
    rx = layer(hx, w2, b2_ref)
    rs = layer(hs, w2, b2_ref)
    reps_ref[...] = rx
    sreps_ref[...] = rs
    ex = layer(rx, wp1, bp1_ref)
    es = layer(rs, wp1, bp1_ref)
    ex = layer(ex, wp2, bp2_ref)
    es = layer(es, wp2, bp2_ref)
    emb_ref[...] = (jnp.dot(ex.astype(bf), wp3, preferred_element_type=jnp.float32)
                    + bp3_ref[...])
    semb_ref[...] = (jnp.dot(es.astype(bf), wp3, preferred_element_type=jnp.float32)
                     + bp3_ref[...])


def kernel(x, x_sim, W1, b1, W2, b2, Wp1, bp1, Wp2, bp2, Wp3, bp3):
    grid = (B // ROWS,)
    row_spec = lambda w: pl.BlockSpec((ROWS, w), lambda i: (i, 0))
    full_spec = lambda r, c: pl.BlockSpec((r, c), lambda i: (0, 0))
    bias_spec = lambda w: pl.BlockSpec((1, w), lambda i: (0, 0))

    out_shapes = (
        jax.ShapeDtypeStruct((B, ENC_OUT), jnp.float32),
        jax.ShapeDtypeStruct((B, EMB), jnp.float32),
        jax.ShapeDtypeStruct((B, ENC_OUT), jnp.float32),
        jax.ShapeDtypeStruct((B, EMB), jnp.float32),
    )

    reps, emb, sreps, semb = pl.pallas_call(
        _fwd_kernel,
        grid=grid,
        in_specs=[
            row_spec(D), row_spec(D),
            full_spec(D, H1), bias_spec(H1),
            full_spec(H1, ENC_OUT), bias_spec(ENC_OUT),
            full_spec(ENC_OUT, P1), bias_spec(P1),
            full_spec(P1, P2), bias_spec(P2),
            full_spec(P2, EMB), bias_spec(EMB),
        ],
        out_specs=(
            row_spec(ENC_OUT), row_spec(EMB),
            row_spec(ENC_OUT), row_spec(EMB),
        ),
        out_shape=out_shapes,
        compiler_params=pltpu.CompilerParams(
            dimension_semantics=("arbitrary",),
        ),
    )(x, x_sim, W1, b1.reshape(1, H1), W2, b2.reshape(1, ENC_OUT),
      Wp1, bp1.reshape(1, P1), Wp2, bp2.reshape(1, P2),
      Wp3, bp3.reshape(1, EMB))

    return (reps, emb, sreps, semb)
